# baseline (speedup 1.0000x reference)
import functools

import jax
import jax.numpy as jnp
from jax import lax
from jax.experimental import pallas as pl
from jax.experimental.pallas import tpu as pltpu
from jax.experimental.pallas import tpu_sc as plsc

N = 100000
E = 3200000
K = 5
OUT = 16

NC = 2
NS = 16
NW = NC * NS
EPT = E // NW
CHUNK = 400
NCHUNK = EPT // CHUNK
GROUPS = CHUNK // 16
SCAT_ROWS = 5
SCAT_C = CHUNK // SCAT_ROWS
ROWS_PT = 6256
NPAD = NS * ROWS_PT
ZBLK = 136


def _sc_body(src_hbm, dst_hbm, pseudo_hbm, x_hbm, w2_hbm,
             out_hbm, w2, srcbuf, dstbuf, pbuf, xchunk, msgbuf, zbuf, sem,
             xsh, agg):
  c = lax.axis_index("c")
  s = lax.axis_index("s")
  wid = c * NS + s

  pltpu.sync_copy(w2_hbm, w2)

  @pl.when(s == 0)
  def _():
    pltpu.sync_copy(x_hbm, xsh)

  def zrow(i, _):
    zbuf[i, :] = jnp.zeros((16,), jnp.float32)
    return 0
  lax.fori_loop(0, ZBLK, zrow, 0)
  rows0 = s * ROWS_PT
  def zcopy(k, _):
    pltpu.sync_copy(zbuf, agg.at[pl.ds(rows0 + k * ZBLK, ZBLK)])
    return 0
  lax.fori_loop(0, ROWS_PT // ZBLK, zcopy, 0)
  plsc.subcore_barrier()

  iota = lax.iota(jnp.int32, 16)
  ebase = wid * EPT

  def chunk_body(j, _):
    off = ebase + j * CHUNK
    pltpu.sync_copy(src_hbm.at[pl.ds(off, CHUNK)], srcbuf)
    for r in range(SCAT_ROWS):
      pltpu.sync_copy(dst_hbm.at[pl.ds(off + r * SCAT_C, SCAT_C)],
                      dstbuf.at[r])
    pltpu.sync_copy(pseudo_hbm.at[pl.ds(off, CHUNK), :], pbuf)
    handles = [
        pltpu.async_copy(xsh.at[srcbuf.at[pl.ds(r * SCAT_C, SCAT_C)]],
                         xchunk.at[pl.ds(r * SCAT_C, SCAT_C)], sem)
        for r in range(SCAT_ROWS)
    ]
    for h in handles:
      h.wait()

    def group(i, _):
      base = i * 16
      e = base + iota
      x_v = xchunk[pl.ds(base, 16)]
      p0 = plsc.load_gather(pbuf, [e, jnp.zeros((16,), jnp.int32)]) * (K - 1.0)
      p1 = plsc.load_gather(pbuf, [e, jnp.full((16,), 1, jnp.int32)]) * (K - 1.0)
      p2 = plsc.load_gather(pbuf, [e, jnp.full((16,), 2, jnp.int32)]) * (K - 1.0)
      lo0 = jnp.minimum(p0.astype(jnp.int32), K - 2)
      lo1 = jnp.minimum(p1.astype(jnp.int32), K - 2)
      lo2 = jnp.minimum(p2.astype(jnp.int32), K - 2)
      f0 = p0 - lo0.astype(jnp.float32)
      f1 = p1 - lo1.astype(jnp.float32)
      f2 = p2 - lo2.astype(jnp.float32)
      g0 = 1.0 - f0
      g1 = 1.0 - f1
      g2 = 1.0 - f2
      cellw = (lo0 + 5 * lo1 + 25 * lo2) * 16
      msgs = [jnp.zeros((16,), jnp.float32) for _ in range(OUT)]
      for bits in range(8):
        dx, dy, dz = bits & 1, (bits >> 1) & 1, (bits >> 2) & 1
        b = ((f0 if dx else g0) * (f1 if dy else g1) * (f2 if dz else g2))
        bx = b * x_v
        widx = cellw + (dx + 5 * dy + 25 * dz) * 16
        for o in range(OUT):
          w = plsc.load_gather(w2, [widx + o])
          msgs[o] = msgs[o] + w * bx
      for o in range(OUT):
        plsc.store_scatter(msgbuf, [e, jnp.full((16,), o, jnp.int32)],
                           msgs[o])
      return 0
    lax.fori_loop(0, GROUPS, group, 0)

    for r in range(SCAT_ROWS):
      pltpu.sync_copy(msgbuf.at[pl.ds(r * SCAT_C, SCAT_C)],
                      agg.at[dstbuf.at[r]], add=True)
    return 0
  lax.fori_loop(0, NCHUNK, chunk_body, 0)

  plsc.subcore_barrier()
  pltpu.sync_copy(agg.at[pl.ds(rows0, ROWS_PT)],
                  out_hbm.at[c].at[pl.ds(rows0, ROWS_PT)])


@jax.jit
def _sc_aggregate(src, dst2d, pseudo_flat, x_flat, w2_flat):
  mesh = plsc.VectorSubcoreMesh(core_axis_name="c", subcore_axis_name="s")
  f = pl.kernel(
      _sc_body,
      out_type=jax.ShapeDtypeStruct((NC, NPAD, OUT), jnp.float32),
      mesh=mesh,
      scratch_types=[
          pltpu.VMEM((K ** 3 * OUT,), jnp.float32),
          pltpu.VMEM((CHUNK,), jnp.int32),
          pltpu.VMEM((SCAT_ROWS, SCAT_C), jnp.int32),
          pltpu.VMEM((CHUNK, 3), jnp.float32),
          pltpu.VMEM((CHUNK,), jnp.float32),
          pltpu.VMEM((CHUNK, OUT), jnp.float32),
          pltpu.VMEM((ZBLK, OUT), jnp.float32),
          pltpu.SemaphoreType.DMA,
          pltpu.VMEM_SHARED((N,), jnp.float32),
          pltpu.VMEM_SHARED((NPAD, OUT), jnp.float32),
      ],
      compiler_params=pltpu.CompilerParams(needs_layout_passes=False,
                                           use_tc_tiling_on_sc=False),
  )
  return f(src, dst2d, pseudo_flat, x_flat, w2_flat)


def _head_body(p0_ref, p1_ref, x_ref, wr_ref, b_ref, lw_ref, lb_ref, o_ref):
  h = p0_ref[...] + p1_ref[...] + x_ref[...] * wr_ref[...] + b_ref[...]
  h = jnp.where(h > 0, h, jnp.exp(jnp.minimum(h, 0.0)) - 1.0)
  q = jnp.dot(h, lw_ref[...], preferred_element_type=jnp.float32) + lb_ref[...]
  sq = jnp.sum(q * q, axis=-1, keepdims=True)
  o_ref[...] = q / (jnp.sqrt(sq) + 1e-4)


@jax.jit
def _head(p0, p1, x, w_root, bias, lin_w, lin_b):
  blk = 2000
  grid = (N // blk,)
  return pl.pallas_call(
      _head_body,
      grid=grid,
      in_specs=[
          pl.BlockSpec((blk, OUT), lambda i: (i, 0)),
          pl.BlockSpec((blk, OUT), lambda i: (i, 0)),
          pl.BlockSpec((blk, 1), lambda i: (i, 0)),
          pl.BlockSpec((1, OUT), lambda i: (0, 0)),
          pl.BlockSpec((1, OUT), lambda i: (0, 0)),
          pl.BlockSpec((OUT, 4), lambda i: (0, 0)),
          pl.BlockSpec((1, 4), lambda i: (0, 0)),
      ],
      out_specs=pl.BlockSpec((blk, 4), lambda i: (i, 0)),
      out_shape=jax.ShapeDtypeStruct((N, 4), jnp.float32),
  )(p0, p1, x, w_root, bias, lin_w, lin_b)


def kernel(x, edge_index, pseudo, W, W_root, bias, lin_W, lin_b):
  src = edge_index[0].astype(jnp.int32)
  dst2d = edge_index[1].astype(jnp.int32)
  pseudo_flat = pseudo
  x_flat = x.reshape(-1)
  w2_flat = W.reshape(-1)
  partials = _sc_aggregate(src, dst2d, pseudo_flat, x_flat, w2_flat)
  out = _head(partials[0, :N], partials[1, :N], x,
              W_root.reshape(1, OUT), bias.reshape(1, OUT),
              lin_W, lin_b.reshape(1, 4))
  return out.reshape(N, 1, 4)


# speedup vs baseline: 2.6978x; 2.6978x over previous
import functools

import jax
import jax.numpy as jnp
from jax import lax
from jax.experimental import pallas as pl
from jax.experimental.pallas import tpu as pltpu
from jax.experimental.pallas import tpu_sc as plsc

N = 100000
E = 3200000
K = 5
OUT = 16

NC = 2
NS = 16
NW = NC * NS
EPT = E // NW
CHUNK = 400
NCHUNK = EPT // CHUNK
GROUPS = CHUNK // 16
SCAT_ROWS = 5
SCAT_C = CHUNK // SCAT_ROWS
ROWS_PT = 6256
NPAD = NS * ROWS_PT
ZBLK = 136


def _sc_body(src_hbm, dst_hbm, cellw_hbm, f0_hbm, f1_hbm, f2_hbm,
             x_hbm, w2_hbm,
             out_hbm, w2, srcbuf, dstbuf, cwbuf, fbuf, xchunk, msgbuf, sem,
             xsh, agg):
  c = lax.axis_index("c")
  s = lax.axis_index("s")
  wid = c * NS + s

  pltpu.sync_copy(w2_hbm, w2)

  @pl.when(s == 0)
  def _():
    pltpu.sync_copy(x_hbm, xsh)

  def zrow(i, _):
    msgbuf[i, :] = jnp.zeros((16,), jnp.float32)
    return 0
  lax.fori_loop(0, ZBLK, zrow, 0)
  rows0 = s * ROWS_PT
  def zcopy(k, _):
    pltpu.sync_copy(msgbuf.at[pl.ds(0, ZBLK)],
                    agg.at[pl.ds(rows0 + k * ZBLK, ZBLK)])
    return 0
  lax.fori_loop(0, ROWS_PT // ZBLK, zcopy, 0)
  plsc.subcore_barrier()

  iota = lax.iota(jnp.int32, 16)
  ebase = wid * EPT

  def chunk_body(j, _):
    off = ebase + j * CHUNK
    pltpu.sync_copy(src_hbm.at[pl.ds(off, CHUNK)], srcbuf)
    for r in range(SCAT_ROWS):
      pltpu.sync_copy(dst_hbm.at[pl.ds(off + r * SCAT_C, SCAT_C)],
                      dstbuf.at[r])
    pltpu.sync_copy(cellw_hbm.at[pl.ds(off, CHUNK)], cwbuf)
    pltpu.sync_copy(f0_hbm.at[pl.ds(off, CHUNK)], fbuf.at[0])
    pltpu.sync_copy(f1_hbm.at[pl.ds(off, CHUNK)], fbuf.at[1])
    pltpu.sync_copy(f2_hbm.at[pl.ds(off, CHUNK)], fbuf.at[2])
    handles = [
        pltpu.async_copy(xsh.at[srcbuf.at[pl.ds(r * SCAT_C, SCAT_C)]],
                         xchunk.at[pl.ds(r * SCAT_C, SCAT_C)], sem)
        for r in range(SCAT_ROWS)
    ]
    for h in handles:
      h.wait()

    def group(i, _):
      base = i * 16
      e = base + iota
      x_v = xchunk[pl.ds(base, 16)]
      cellw = cwbuf[pl.ds(base, 16)]
      f0 = fbuf[0, pl.ds(base, 16)]
      f1 = fbuf[1, pl.ds(base, 16)]
      f2 = fbuf[2, pl.ds(base, 16)]
      g0 = 1.0 - f0
      g1 = 1.0 - f1
      g2 = 1.0 - f2
      msgs = [jnp.zeros((16,), jnp.float32) for _ in range(OUT)]
      for bits in range(8):
        dx, dy, dz = bits & 1, (bits >> 1) & 1, (bits >> 2) & 1
        b = ((f0 if dx else g0) * (f1 if dy else g1) * (f2 if dz else g2))
        bx = b * x_v
        widx = cellw + (dx + 5 * dy + 25 * dz) * 16
        for o in range(OUT):
          w = plsc.load_gather(w2, [widx + o])
          msgs[o] = msgs[o] + w * bx
      for o in range(OUT):
        plsc.store_scatter(msgbuf, [e, jnp.full((16,), o, jnp.int32)],
                           msgs[o])
      return 0
    lax.fori_loop(0, GROUPS, group, 0)

    for r in range(SCAT_ROWS):
      pltpu.sync_copy(msgbuf.at[pl.ds(r * SCAT_C, SCAT_C)],
                      agg.at[dstbuf.at[r]], add=True)
    return 0
  lax.fori_loop(0, NCHUNK, chunk_body, 0)

  plsc.subcore_barrier()
  pltpu.sync_copy(agg.at[pl.ds(rows0, ROWS_PT)],
                  out_hbm.at[c].at[pl.ds(rows0, ROWS_PT)])


def _sc_aggregate(src, dst, cellw, f0, f1, f2, x_flat, w2_flat):
  mesh = plsc.VectorSubcoreMesh(core_axis_name="c", subcore_axis_name="s")
  f = pl.kernel(
      _sc_body,
      out_type=jax.ShapeDtypeStruct((NC, NPAD, OUT), jnp.float32),
      mesh=mesh,
      scratch_types=[
          pltpu.VMEM((K ** 3 * OUT,), jnp.float32),
          pltpu.VMEM((CHUNK,), jnp.int32),
          pltpu.VMEM((SCAT_ROWS, SCAT_C), jnp.int32),
          pltpu.VMEM((CHUNK,), jnp.int32),
          pltpu.VMEM((3, CHUNK), jnp.float32),
          pltpu.VMEM((CHUNK,), jnp.float32),
          pltpu.VMEM((CHUNK, OUT), jnp.float32),
          pltpu.SemaphoreType.DMA,
          pltpu.VMEM_SHARED((NXPAD,), jnp.float32),
          pltpu.VMEM_SHARED((NPAD, OUT), jnp.float32),
      ],
      compiler_params=pltpu.CompilerParams(needs_layout_passes=False,
                                           use_tc_tiling_on_sc=False),
  )
  return f(src, dst, cellw, f0, f1, f2, x_flat, w2_flat)


EB = 25600
PB = 5120
XB = 10240
NXPAD = 102400


def _edge_prep_body(ei_ref, src_ref, dst_ref):
  src_ref[...] = ei_ref[0, :]
  dst_ref[...] = ei_ref[1, :]


def _pseudo_prep_body(p_ref, cw_ref, f0_ref, f1_ref, f2_ref):
  pt = p_ref[...].T
  cw = jnp.zeros((PB,), jnp.int32)
  fs = [f0_ref, f1_ref, f2_ref]
  strides = (1, K, K * K)
  for d in range(3):
    pd = pt[d, :] * (K - 1.0)
    lo = jnp.minimum(pd.astype(jnp.int32), K - 2)
    fs[d][...] = pd - lo.astype(jnp.float32)
    cw = cw + lo * (strides[d] * OUT)
  cw_ref[...] = cw


def _x_prep_body(x_ref, o_ref):
  o_ref[...] = x_ref[...].T[0, :]


def _prep(edge_index, pseudo, x):
  src, dst = pl.pallas_call(
      _edge_prep_body,
      grid=(E // EB,),
      in_specs=[pl.BlockSpec((2, EB), lambda i: (0, i))],
      out_specs=[pl.BlockSpec((EB,), lambda i: (i,)),
                 pl.BlockSpec((EB,), lambda i: (i,))],
      out_shape=[jax.ShapeDtypeStruct((E,), jnp.int32),
                 jax.ShapeDtypeStruct((E,), jnp.int32)],
  )(edge_index)
  cellw, f0, f1, f2 = pl.pallas_call(
      _pseudo_prep_body,
      grid=(E // PB,),
      in_specs=[pl.BlockSpec((PB, 3), lambda i: (i, 0))],
      out_specs=[pl.BlockSpec((PB,), lambda i: (i,)) for _ in range(4)],
      out_shape=[jax.ShapeDtypeStruct((E,), jnp.int32)] +
                [jax.ShapeDtypeStruct((E,), jnp.float32) for _ in range(3)],
  )(pseudo)
  x_flat = pl.pallas_call(
      _x_prep_body,
      grid=(NXPAD // XB,),
      in_specs=[pl.BlockSpec((XB, 1), lambda i: (i, 0))],
      out_specs=pl.BlockSpec((XB,), lambda i: (i,)),
      out_shape=jax.ShapeDtypeStruct((NXPAD,), jnp.float32),
  )(x)
  return src, dst, cellw, f0, f1, f2, x_flat


def _head_body(p0_ref, p1_ref, x_ref, wr_ref, b_ref, lw_ref, lb_ref, o_ref):
  h = p0_ref[...] + p1_ref[...] + x_ref[...] * wr_ref[...] + b_ref[...]
  h = jnp.where(h > 0, h, jnp.exp(jnp.minimum(h, 0.0)) - 1.0)
  q = jnp.dot(h, lw_ref[...], preferred_element_type=jnp.float32) + lb_ref[...]
  sq = jnp.sum(q * q, axis=-1, keepdims=True)
  o_ref[...] = q / (jnp.sqrt(sq) + 1e-4)


def _head(p0, p1, x, w_root, bias, lin_w, lin_b):
  blk = 2000
  grid = (N // blk,)
  return pl.pallas_call(
      _head_body,
      grid=grid,
      in_specs=[
          pl.BlockSpec((blk, OUT), lambda i: (i, 0)),
          pl.BlockSpec((blk, OUT), lambda i: (i, 0)),
          pl.BlockSpec((blk, 1), lambda i: (i, 0)),
          pl.BlockSpec((1, OUT), lambda i: (0, 0)),
          pl.BlockSpec((1, OUT), lambda i: (0, 0)),
          pl.BlockSpec((OUT, 4), lambda i: (0, 0)),
          pl.BlockSpec((1, 4), lambda i: (0, 0)),
      ],
      out_specs=pl.BlockSpec((blk, 4), lambda i: (i, 0)),
      out_shape=jax.ShapeDtypeStruct((N, 4), jnp.float32),
  )(p0, p1, x, w_root, bias, lin_w, lin_b)


@jax.jit
def _run(x, edge_index, pseudo, W, W_root, bias, lin_W, lin_b):
  src, dst, cellw, f0, f1, f2, x_flat = _prep(edge_index, pseudo, x)
  w2_flat = W.reshape(-1)
  partials = _sc_aggregate(src, dst, cellw, f0, f1, f2, x_flat, w2_flat)
  out = _head(partials[0, :N], partials[1, :N], x,
              W_root.reshape(1, OUT), bias.reshape(1, OUT),
              lin_W, lin_b.reshape(1, 4))
  return out.reshape(N, 1, 4)


def kernel(x, edge_index, pseudo, W, W_root, bias, lin_W, lin_b):
  return _run(x, edge_index, pseudo, W, W_root, bias, lin_W, lin_b)


# speedup vs baseline: 3.5139x; 1.3025x over previous
import functools

import jax
import jax.numpy as jnp
from jax import lax
from jax.experimental import pallas as pl
from jax.experimental.pallas import tpu as pltpu
from jax.experimental.pallas import tpu_sc as plsc

N = 100000
E = 3200000
K = 5
OUT = 16

NC = 2
NS = 16
NW = NC * NS
EPT = E // NW
CHUNK = 400
NCHUNK = EPT // CHUNK
GROUPS = CHUNK // 16
SCAT_ROWS = 5
SCAT_C = CHUNK // SCAT_ROWS
ROWS_PT = 6256
NPAD = NS * ROWS_PT
ZBLK = 136


def _sc_body(src_hbm, dst_hbm, cellw_hbm, f0_hbm, f1_hbm, f2_hbm,
             x_hbm, w2_hbm, out_hbm,
             w2, srcb0, srcb1, dstb0, dstb1, cwb0, cwb1, fb0, fb1,
             xc0, xc1, msg0, msg1,
             in_sem0, in_sem1, x_sem, sc_sem0, sc_sem1,
             xsh, agg):
  c = lax.axis_index("c")
  s = lax.axis_index("s")
  wid = c * NS + s
  srcb = (srcb0, srcb1)
  dstb = (dstb0, dstb1)
  cwb = (cwb0, cwb1)
  fb = (fb0, fb1)
  xc = (xc0, xc1)
  msg = (msg0, msg1)
  in_sem = (in_sem0, in_sem1)
  sc_sem = (sc_sem0, sc_sem1)

  pltpu.sync_copy(w2_hbm, w2)

  @pl.when(s == 0)
  def _():
    pltpu.sync_copy(x_hbm.at[pl.ds(0, N)], xsh)

  def zrow(i, _):
    msg0[i, :] = jnp.zeros((16,), jnp.float32)
    return 0
  lax.fori_loop(0, ZBLK, zrow, 0)
  rows0 = s * ROWS_PT
  def zcopy(k, _):
    pltpu.sync_copy(msg0.at[pl.ds(0, ZBLK)],
                    agg.at[pl.ds(rows0 + k * ZBLK, ZBLK)])
    return 0
  lax.fori_loop(0, ROWS_PT // ZBLK, zcopy, 0)
  plsc.subcore_barrier()

  iota = lax.iota(jnp.int32, 16)
  ebase = wid * EPT

  def load_handles(j, b, make):
    off = ebase + j * CHUNK
    f = pltpu.make_async_copy if make else (
        lambda a, d, m: pltpu.async_copy(a, d, m))
    hs = [f(src_hbm.at[pl.ds(off, CHUNK)], srcb[b], in_sem[b]),
          f(cellw_hbm.at[pl.ds(off, CHUNK)], cwb[b], in_sem[b]),
          f(f0_hbm.at[pl.ds(off, CHUNK)], fb[b].at[0], in_sem[b]),
          f(f1_hbm.at[pl.ds(off, CHUNK)], fb[b].at[1], in_sem[b]),
          f(f2_hbm.at[pl.ds(off, CHUNK)], fb[b].at[2], in_sem[b])]
    hs += [f(dst_hbm.at[pl.ds(off + r * SCAT_C, SCAT_C)], dstb[b].at[r],
             in_sem[b]) for r in range(SCAT_ROWS)]
    return hs

  def scat_handles(b, make):
    if make:
      return [pltpu.make_async_copy(msg[b].at[pl.ds(r * SCAT_C, SCAT_C)],
                                    agg.at[dstb[b].at[r]], sc_sem[b])
              for r in range(SCAT_ROWS)]
    return [pltpu.async_copy(msg[b].at[pl.ds(r * SCAT_C, SCAT_C)],
                             agg.at[dstb[b].at[r]], sc_sem[b], add=True)
            for r in range(SCAT_ROWS)]

  def compute(b):
    def group(i, _):
      base = i * 16
      e = base + iota
      x_v = xc[b][pl.ds(base, 16)]
      cellw = cwb[b][pl.ds(base, 16)]
      f0 = fb[b][0, pl.ds(base, 16)]
      f1 = fb[b][1, pl.ds(base, 16)]
      f2 = fb[b][2, pl.ds(base, 16)]
      g0 = 1.0 - f0
      g1 = 1.0 - f1
      g2 = 1.0 - f2
      msgs = [jnp.zeros((16,), jnp.float32) for _ in range(OUT)]
      for bits in range(8):
        dx, dy, dz = bits & 1, (bits >> 1) & 1, (bits >> 2) & 1
        bv = ((f0 if dx else g0) * (f1 if dy else g1) * (f2 if dz else g2))
        bx = bv * x_v
        widx = cellw + (dx + 5 * dy + 25 * dz) * 16
        for o in range(OUT):
          w = plsc.load_gather(w2, [widx + o])
          msgs[o] = msgs[o] + w * bx
      for o in range(OUT):
        plsc.store_scatter(msg[b], [e, jnp.full((16,), o, jnp.int32)],
                           msgs[o])
      return 0
    lax.fori_loop(0, GROUPS, group, 0)

  load_handles(0, 0, False)

  def outer(jo, _):
    for b in range(2):
      j = 2 * jo + b
      nb = 1 - b
      for h in load_handles(j, b, True):
        h.wait()
      xh = [pltpu.async_copy(
          xsh.at[srcb[b].at[pl.ds(r * SCAT_C, SCAT_C)]],
          xc[b].at[pl.ds(r * SCAT_C, SCAT_C)], x_sem)
          for r in range(SCAT_ROWS)]
      for h in xh:
        h.wait()

      @pl.when(j >= 1)
      def _():
        for h in scat_handles(nb, True):
          h.wait()

      @pl.when(j + 1 < NCHUNK)
      def _():
        load_handles(j + 1, nb, False)

      compute(b)
      scat_handles(b, False)
    return 0
  lax.fori_loop(0, NCHUNK // 2, outer, 0)
  for h in scat_handles(1, True):
    h.wait()

  plsc.subcore_barrier()
  pltpu.sync_copy(agg.at[pl.ds(rows0, ROWS_PT)],
                  out_hbm.at[c].at[pl.ds(rows0, ROWS_PT)])


def _sc_aggregate(src, dst, cellw, f0, f1, f2, x_flat, w2_flat):
  mesh = plsc.VectorSubcoreMesh(core_axis_name="c", subcore_axis_name="s")
  f = pl.kernel(
      _sc_body,
      out_type=jax.ShapeDtypeStruct((NC, NPAD, OUT), jnp.float32),
      mesh=mesh,
      scratch_types=(
          [pltpu.VMEM((K ** 3 * OUT,), jnp.float32)] +
          [pltpu.VMEM((CHUNK,), jnp.int32)] * 2 +
          [pltpu.VMEM((SCAT_ROWS, SCAT_C), jnp.int32)] * 2 +
          [pltpu.VMEM((CHUNK,), jnp.int32)] * 2 +
          [pltpu.VMEM((3, CHUNK), jnp.float32)] * 2 +
          [pltpu.VMEM((CHUNK,), jnp.float32)] * 2 +
          [pltpu.VMEM((CHUNK, OUT), jnp.float32)] * 2 +
          [pltpu.SemaphoreType.DMA] * 5 +
          [pltpu.VMEM_SHARED((N,), jnp.float32),
           pltpu.VMEM_SHARED((NPAD, OUT), jnp.float32)]
      ),
      compiler_params=pltpu.CompilerParams(needs_layout_passes=False,
                                           use_tc_tiling_on_sc=False),
  )
  return f(src, dst, cellw, f0, f1, f2, x_flat, w2_flat)


EB = 25600
PB = 5120
XB = 10240
NXPAD = 102400


def _edge_prep_body(ei_ref, src_ref, dst_ref):
  src_ref[...] = ei_ref[0, :]
  dst_ref[...] = ei_ref[1, :]


def _pseudo_prep_body(p_ref, cw_ref, f0_ref, f1_ref, f2_ref):
  pt = p_ref[...].T
  cw = jnp.zeros((PB,), jnp.int32)
  fs = [f0_ref, f1_ref, f2_ref]
  strides = (1, K, K * K)
  for d in range(3):
    pd = pt[d, :] * (K - 1.0)
    lo = jnp.minimum(pd.astype(jnp.int32), K - 2)
    fs[d][...] = pd - lo.astype(jnp.float32)
    cw = cw + lo * (strides[d] * OUT)
  cw_ref[...] = cw


def _x_prep_body(x_ref, o_ref):
  o_ref[...] = x_ref[...].T[0, :]


def _prep(edge_index, pseudo, x):
  src, dst = pl.pallas_call(
      _edge_prep_body,
      grid=(E // EB,),
      in_specs=[pl.BlockSpec((2, EB), lambda i: (0, i))],
      out_specs=[pl.BlockSpec((EB,), lambda i: (i,)),
                 pl.BlockSpec((EB,), lambda i: (i,))],
      out_shape=[jax.ShapeDtypeStruct((E,), jnp.int32),
                 jax.ShapeDtypeStruct((E,), jnp.int32)],
  )(edge_index)
  cellw, f0, f1, f2 = pl.pallas_call(
      _pseudo_prep_body,
      grid=(E // PB,),
      in_specs=[pl.BlockSpec((PB, 3), lambda i: (i, 0))],
      out_specs=[pl.BlockSpec((PB,), lambda i: (i,)) for _ in range(4)],
      out_shape=[jax.ShapeDtypeStruct((E,), jnp.int32)] +
                [jax.ShapeDtypeStruct((E,), jnp.float32) for _ in range(3)],
  )(pseudo)
  x_flat = pl.pallas_call(
      _x_prep_body,
      grid=(NXPAD // XB,),
      in_specs=[pl.BlockSpec((XB, 1), lambda i: (i, 0))],
      out_specs=pl.BlockSpec((XB,), lambda i: (i,)),
      out_shape=jax.ShapeDtypeStruct((NXPAD,), jnp.float32),
  )(x)
  return src, dst, cellw, f0, f1, f2, x_flat


def _head_body(p0_ref, p1_ref, x_ref, wr_ref, b_ref, lw_ref, lb_ref, o_ref):
  h = p0_ref[...] + p1_ref[...] + x_ref[...] * wr_ref[...] + b_ref[...]
  h = jnp.where(h > 0, h, jnp.exp(jnp.minimum(h, 0.0)) - 1.0)
  q = jnp.dot(h, lw_ref[...], preferred_element_type=jnp.float32) + lb_ref[...]
  sq = jnp.sum(q * q, axis=-1, keepdims=True)
  o_ref[...] = q / (jnp.sqrt(sq) + 1e-4)


def _head(p0, p1, x, w_root, bias, lin_w, lin_b):
  blk = 2000
  grid = (N // blk,)
  return pl.pallas_call(
      _head_body,
      grid=grid,
      in_specs=[
          pl.BlockSpec((blk, OUT), lambda i: (i, 0)),
          pl.BlockSpec((blk, OUT), lambda i: (i, 0)),
          pl.BlockSpec((blk, 1), lambda i: (i, 0)),
          pl.BlockSpec((1, OUT), lambda i: (0, 0)),
          pl.BlockSpec((1, OUT), lambda i: (0, 0)),
          pl.BlockSpec((OUT, 4), lambda i: (0, 0)),
          pl.BlockSpec((1, 4), lambda i: (0, 0)),
      ],
      out_specs=pl.BlockSpec((blk, 4), lambda i: (i, 0)),
      out_shape=jax.ShapeDtypeStruct((N, 4), jnp.float32),
  )(p0, p1, x, w_root, bias, lin_w, lin_b)


@jax.jit
def _run(x, edge_index, pseudo, W, W_root, bias, lin_W, lin_b):
  src, dst, cellw, f0, f1, f2, x_flat = _prep(edge_index, pseudo, x)
  w2_flat = W.reshape(-1)
  partials = _sc_aggregate(src, dst, cellw, f0, f1, f2, x_flat, w2_flat)
  out = _head(partials[0, :N], partials[1, :N], x,
              W_root.reshape(1, OUT), bias.reshape(1, OUT),
              lin_W, lin_b.reshape(1, 4))
  return out.reshape(N, 1, 4)


def kernel(x, edge_index, pseudo, W, W_root, bias, lin_W, lin_b):
  return _run(x, edge_index, pseudo, W, W_root, bias, lin_W, lin_b)


# speedup vs baseline: 3.8269x; 1.0891x over previous
import functools

import jax
import jax.numpy as jnp
from jax import lax
from jax.experimental import pallas as pl
from jax.experimental.pallas import tpu as pltpu
from jax.experimental.pallas import tpu_sc as plsc

N = 100000
E = 3200000
K = 5
OUT = 16

NC = 2
NS = 16
NW = NC * NS
EPT = E // NW
CHUNK = 400
NCHUNK = EPT // CHUNK
GROUPS = CHUNK // 16
SCAT_ROWS = 5
SCAT_C = CHUNK // SCAT_ROWS
ROWS_PT = 6256
NPAD = NS * ROWS_PT
ZBLK = 136


def _sc_body(src_hbm, dst_hbm, cellw_hbm, f0_hbm, f1_hbm, f2_hbm,
             x_hbm, w2_hbm, out_hbm,
             w2, srcb0, srcb1, dstb0, dstb1, cwb0, cwb1, fb0, fb1,
             xc0, xc1, msg0, msg1,
             in_sem0, in_sem1, x_sem, sc_sem0, sc_sem1,
             xsh, agg):
  c = lax.axis_index("c")
  s = lax.axis_index("s")
  wid = c * NS + s
  srcb = (srcb0, srcb1)
  dstb = (dstb0, dstb1)
  cwb = (cwb0, cwb1)
  fb = (fb0, fb1)
  xc = (xc0, xc1)
  msg = (msg0, msg1)
  in_sem = (in_sem0, in_sem1)
  sc_sem = (sc_sem0, sc_sem1)

  pltpu.sync_copy(w2_hbm, w2)

  @pl.when(s == 0)
  def _():
    pltpu.sync_copy(x_hbm.at[pl.ds(0, N)], xsh)

  def zrow(i, _):
    msg0[i, :] = jnp.zeros((16,), jnp.float32)
    return 0
  lax.fori_loop(0, ZBLK, zrow, 0)
  rows0 = s * ROWS_PT
  def zcopy(k, _):
    pltpu.sync_copy(msg0.at[pl.ds(0, ZBLK)],
                    agg.at[pl.ds(rows0 + k * ZBLK, ZBLK)])
    return 0
  lax.fori_loop(0, ROWS_PT // ZBLK, zcopy, 0)
  plsc.subcore_barrier()

  iota = lax.iota(jnp.int32, 16)
  ebase = wid * EPT

  def load_handles(j, b, make):
    off = ebase + j * CHUNK
    f = pltpu.make_async_copy if make else (
        lambda a, d, m: pltpu.async_copy(a, d, m))
    hs = [f(src_hbm.at[pl.ds(off, CHUNK)], srcb[b], in_sem[b]),
          f(cellw_hbm.at[pl.ds(off, CHUNK)], cwb[b], in_sem[b]),
          f(f0_hbm.at[pl.ds(off, CHUNK)], fb[b].at[0], in_sem[b]),
          f(f1_hbm.at[pl.ds(off, CHUNK)], fb[b].at[1], in_sem[b]),
          f(f2_hbm.at[pl.ds(off, CHUNK)], fb[b].at[2], in_sem[b])]
    hs += [f(dst_hbm.at[pl.ds(off + r * SCAT_C, SCAT_C)], dstb[b].at[r],
             in_sem[b]) for r in range(SCAT_ROWS)]
    return hs

  def scat_handles(b, make):
    if make:
      return [pltpu.make_async_copy(msg[b].at[pl.ds(r * SCAT_C, SCAT_C)],
                                    agg.at[dstb[b].at[r]], sc_sem[b])
              for r in range(SCAT_ROWS)]
    return [pltpu.async_copy(msg[b].at[pl.ds(r * SCAT_C, SCAT_C)],
                             agg.at[dstb[b].at[r]], sc_sem[b], add=True)
            for r in range(SCAT_ROWS)]

  def compute(b):
    def group(i, _):
      base = i * 16
      e = base + iota
      x_v = xc[b][pl.ds(base, 16)]
      cellw = cwb[b][pl.ds(base, 16)]
      f0 = fb[b][0, pl.ds(base, 16)]
      f1 = fb[b][1, pl.ds(base, 16)]
      f2 = fb[b][2, pl.ds(base, 16)]
      g0 = 1.0 - f0
      g1 = 1.0 - f1
      g2 = 1.0 - f2
      msgs = [jnp.zeros((16,), jnp.float32) for _ in range(OUT)]
      for bits in range(8):
        dx, dy, dz = bits & 1, (bits >> 1) & 1, (bits >> 2) & 1
        bv = ((f0 if dx else g0) * (f1 if dy else g1) * (f2 if dz else g2))
        bx = bv * x_v
        widx = cellw + (dx + 5 * dy + 25 * dz) * 16
        for o in range(OUT):
          w = plsc.load_gather(w2, [widx + o])
          msgs[o] = msgs[o] + w * bx
      for o in range(OUT):
        plsc.store_scatter(msg[b], [e, jnp.full((16,), o, jnp.int32)],
                           msgs[o])
      return 0
    lax.fori_loop(0, GROUPS, group, 0)

  load_handles(0, 0, False)

  def outer(jo, _):
    for b in range(2):
      j = 2 * jo + b
      nb = 1 - b
      for h in load_handles(j, b, True):
        h.wait()
      xh = [pltpu.async_copy(
          xsh.at[srcb[b].at[pl.ds(r * SCAT_C, SCAT_C)]],
          xc[b].at[pl.ds(r * SCAT_C, SCAT_C)], x_sem)
          for r in range(SCAT_ROWS)]

      @pl.when(j >= 1)
      def _():
        for h in scat_handles(nb, True):
          h.wait()

      @pl.when(j + 1 < NCHUNK)
      def _():
        load_handles(j + 1, nb, False)

      for h in xh:
        h.wait()
      compute(b)
      scat_handles(b, False)
    return 0
  lax.fori_loop(0, NCHUNK // 2, outer, 0)
  for h in scat_handles(1, True):
    h.wait()

  plsc.subcore_barrier()
  pltpu.sync_copy(agg.at[pl.ds(rows0, ROWS_PT)],
                  out_hbm.at[c].at[pl.ds(rows0, ROWS_PT)])


def _sc_aggregate(src, dst, cellw, f0, f1, f2, x_flat, w2_flat):
  mesh = plsc.VectorSubcoreMesh(core_axis_name="c", subcore_axis_name="s")
  f = pl.kernel(
      _sc_body,
      out_type=jax.ShapeDtypeStruct((NC, NPAD, OUT), jnp.float32),
      mesh=mesh,
      scratch_types=(
          [pltpu.VMEM((K ** 3 * OUT,), jnp.float32)] +
          [pltpu.VMEM((CHUNK,), jnp.int32)] * 2 +
          [pltpu.VMEM((SCAT_ROWS, SCAT_C), jnp.int32)] * 2 +
          [pltpu.VMEM((CHUNK,), jnp.int32)] * 2 +
          [pltpu.VMEM((3, CHUNK), jnp.float32)] * 2 +
          [pltpu.VMEM((CHUNK,), jnp.float32)] * 2 +
          [pltpu.VMEM((CHUNK, OUT), jnp.float32)] * 2 +
          [pltpu.SemaphoreType.DMA] * 5 +
          [pltpu.VMEM_SHARED((N,), jnp.float32),
           pltpu.VMEM_SHARED((NPAD, OUT), jnp.float32)]
      ),
      compiler_params=pltpu.CompilerParams(needs_layout_passes=False,
                                           use_tc_tiling_on_sc=False),
  )
  return f(src, dst, cellw, f0, f1, f2, x_flat, w2_flat)


EB = 25600
XB = 10240
NXPAD = 102400


def _prep_body(ei_ref, p_ref, src_ref, dst_ref, cw_ref,
               f0_ref, f1_ref, f2_ref):
  src_ref[...] = ei_ref[0, :]
  dst_ref[...] = ei_ref[1, :]
  pt = p_ref[...].T
  cw = jnp.zeros((EB,), jnp.int32)
  fs = [f0_ref, f1_ref, f2_ref]
  strides = (1, K, K * K)
  for d in range(3):
    pd = pt[d, :] * (K - 1.0)
    lo = jnp.minimum(pd.astype(jnp.int32), K - 2)
    fs[d][...] = pd - lo.astype(jnp.float32)
    cw = cw + lo * (strides[d] * OUT)
  cw_ref[...] = cw


def _x_prep_body(x_ref, o_ref):
  o_ref[...] = x_ref[...].T[0, :]


def _prep(edge_index, pseudo, x):
  src, dst, cellw, f0, f1, f2 = pl.pallas_call(
      _prep_body,
      grid=(E // EB,),
      in_specs=[pl.BlockSpec((2, EB), lambda i: (0, i)),
                pl.BlockSpec((EB, 3), lambda i: (i, 0))],
      out_specs=[pl.BlockSpec((EB,), lambda i: (i,)) for _ in range(6)],
      out_shape=[jax.ShapeDtypeStruct((E,), jnp.int32)] * 3 +
                [jax.ShapeDtypeStruct((E,), jnp.float32)] * 3,
  )(edge_index, pseudo)
  x_flat = pl.pallas_call(
      _x_prep_body,
      grid=(NXPAD // XB,),
      in_specs=[pl.BlockSpec((XB, 1), lambda i: (i, 0))],
      out_specs=pl.BlockSpec((XB,), lambda i: (i,)),
      out_shape=jax.ShapeDtypeStruct((NXPAD,), jnp.float32),
  )(x)
  return src, dst, cellw, f0, f1, f2, x_flat


def _head_body(p0_ref, p1_ref, x_ref, wr_ref, b_ref, lw_ref, lb_ref, o_ref):
  h = p0_ref[...] + p1_ref[...] + x_ref[...] * wr_ref[...] + b_ref[...]
  h = jnp.where(h > 0, h, jnp.exp(jnp.minimum(h, 0.0)) - 1.0)
  q = jnp.dot(h, lw_ref[...], preferred_element_type=jnp.float32) + lb_ref[...]
  sq = jnp.sum(q * q, axis=-1, keepdims=True)
  o_ref[...] = q / (jnp.sqrt(sq) + 1e-4)


def _head(p0, p1, x, w_root, bias, lin_w, lin_b):
  blk = 2000
  grid = (N // blk,)
  return pl.pallas_call(
      _head_body,
      grid=grid,
      in_specs=[
          pl.BlockSpec((blk, OUT), lambda i: (i, 0)),
          pl.BlockSpec((blk, OUT), lambda i: (i, 0)),
          pl.BlockSpec((blk, 1), lambda i: (i, 0)),
          pl.BlockSpec((1, OUT), lambda i: (0, 0)),
          pl.BlockSpec((1, OUT), lambda i: (0, 0)),
          pl.BlockSpec((OUT, 4), lambda i: (0, 0)),
          pl.BlockSpec((1, 4), lambda i: (0, 0)),
      ],
      out_specs=pl.BlockSpec((blk, 4), lambda i: (i, 0)),
      out_shape=jax.ShapeDtypeStruct((N, 4), jnp.float32),
  )(p0, p1, x, w_root, bias, lin_w, lin_b)


@jax.jit
def _run(x, edge_index, pseudo, W, W_root, bias, lin_W, lin_b):
  src, dst, cellw, f0, f1, f2, x_flat = _prep(edge_index, pseudo, x)
  w2_flat = W.reshape(-1)
  partials = _sc_aggregate(src, dst, cellw, f0, f1, f2, x_flat, w2_flat)
  out = _head(partials[0, :N], partials[1, :N], x,
              W_root.reshape(1, OUT), bias.reshape(1, OUT),
              lin_W, lin_b.reshape(1, 4))
  return out.reshape(N, 1, 4)


def kernel(x, edge_index, pseudo, W, W_root, bias, lin_W, lin_b):
  return _run(x, edge_index, pseudo, W, W_root, bias, lin_W, lin_b)


# speedup vs baseline: 3.8440x; 1.0045x over previous
import functools

import jax
import jax.numpy as jnp
from jax import lax
from jax.experimental import pallas as pl
from jax.experimental.pallas import tpu as pltpu
from jax.experimental.pallas import tpu_sc as plsc

N = 100000
E = 3200000
K = 5
OUT = 16

NC = 2
NS = 16
NW = NC * NS
EPT = E // NW
CHUNK = 400
NCHUNK = EPT // CHUNK
GROUPS = CHUNK // 16
SCAT_ROWS = 5
SCAT_C = CHUNK // SCAT_ROWS
ROWS_PT = 6256
NPAD = NS * ROWS_PT
ZBLK = 136


def _sc_body(src_hbm, dst_hbm, cellw_hbm, f0_hbm, f1_hbm, f2_hbm,
             x_hbm, w2_hbm, out_hbm,
             w2, srcb0, srcb1, dstb0, dstb1, cwb0, cwb1, fb0, fb1,
             xc0, xc1, msg0, msg1,
             in_sem0, in_sem1, x_sem, sc_sem0, sc_sem1,
             xsh, agg):
  c = lax.axis_index("c")
  s = lax.axis_index("s")
  wid = c * NS + s
  srcb = (srcb0, srcb1)
  dstb = (dstb0, dstb1)
  cwb = (cwb0, cwb1)
  fb = (fb0, fb1)
  xc = (xc0, xc1)
  msg = (msg0, msg1)
  in_sem = (in_sem0, in_sem1)
  sc_sem = (sc_sem0, sc_sem1)

  pltpu.sync_copy(w2_hbm, w2)

  @pl.when(s == 0)
  def _():
    pltpu.sync_copy(x_hbm.at[pl.ds(0, N)], xsh)

  def zrow(i, _):
    msg0[i, :] = jnp.zeros((16,), jnp.float32)
    return 0
  lax.fori_loop(0, ZBLK, zrow, 0)
  rows0 = s * ROWS_PT
  def zcopy(k, _):
    pltpu.sync_copy(msg0.at[pl.ds(0, ZBLK)],
                    agg.at[pl.ds(rows0 + k * ZBLK, ZBLK)])
    return 0
  lax.fori_loop(0, ROWS_PT // ZBLK, zcopy, 0)
  plsc.subcore_barrier()

  iota = lax.iota(jnp.int32, 16)
  ebase = wid * EPT

  def load_handles(j, b, make):
    off = ebase + j * CHUNK
    f = pltpu.make_async_copy if make else (
        lambda a, d, m: pltpu.async_copy(a, d, m))
    hs = [f(src_hbm.at[pl.ds(off, CHUNK)], srcb[b], in_sem[b]),
          f(cellw_hbm.at[pl.ds(off, CHUNK)], cwb[b], in_sem[b]),
          f(f0_hbm.at[pl.ds(off, CHUNK)], fb[b].at[0], in_sem[b]),
          f(f1_hbm.at[pl.ds(off, CHUNK)], fb[b].at[1], in_sem[b]),
          f(f2_hbm.at[pl.ds(off, CHUNK)], fb[b].at[2], in_sem[b])]
    hs += [f(dst_hbm.at[pl.ds(off + r * SCAT_C, SCAT_C)], dstb[b].at[r],
             in_sem[b]) for r in range(SCAT_ROWS)]
    return hs

  def scat_handles(b, make):
    if make:
      return [pltpu.make_async_copy(msg[b].at[pl.ds(r * SCAT_C, SCAT_C)],
                                    agg.at[dstb[b].at[r]], sc_sem[b])
              for r in range(SCAT_ROWS)]
    return [pltpu.async_copy(msg[b].at[pl.ds(r * SCAT_C, SCAT_C)],
                             agg.at[dstb[b].at[r]], sc_sem[b], add=True)
            for r in range(SCAT_ROWS)]

  def compute(b):
    def group(i):
      base = i * 16
      e = base + iota
      x_v = xc[b][pl.ds(base, 16)]
      cellw = cwb[b][pl.ds(base, 16)]
      f0 = fb[b][0, pl.ds(base, 16)]
      f1 = fb[b][1, pl.ds(base, 16)]
      f2 = fb[b][2, pl.ds(base, 16)]
      g0 = 1.0 - f0
      g1 = 1.0 - f1
      g2 = 1.0 - f2
      msgs = [jnp.zeros((16,), jnp.float32) for _ in range(OUT)]
      for bits in range(8):
        dx, dy, dz = bits & 1, (bits >> 1) & 1, (bits >> 2) & 1
        bv = ((f0 if dx else g0) * (f1 if dy else g1) * (f2 if dz else g2))
        bx = bv * x_v
        widx = cellw + (dx + 5 * dy + 25 * dz) * 16
        for o in range(OUT):
          w = plsc.load_gather(w2, [widx + o])
          msgs[o] = msgs[o] + w * bx
      for o in range(OUT):
        plsc.store_scatter(msg[b], [e, jnp.full((16,), o, jnp.int32)],
                           msgs[o])
    plsc.parallel_loop(0, GROUPS, 1, unroll=2)(group)

  load_handles(0, 0, False)

  def outer(jo, _):
    for b in range(2):
      j = 2 * jo + b
      nb = 1 - b
      for h in load_handles(j, b, True):
        h.wait()
      xh = [pltpu.async_copy(
          xsh.at[srcb[b].at[pl.ds(r * SCAT_C, SCAT_C)]],
          xc[b].at[pl.ds(r * SCAT_C, SCAT_C)], x_sem)
          for r in range(SCAT_ROWS)]

      @pl.when(j >= 1)
      def _():
        for h in scat_handles(nb, True):
          h.wait()

      @pl.when(j + 1 < NCHUNK)
      def _():
        load_handles(j + 1, nb, False)

      for h in xh:
        h.wait()
      compute(b)
      scat_handles(b, False)
    return 0
  lax.fori_loop(0, NCHUNK // 2, outer, 0)
  for h in scat_handles(1, True):
    h.wait()

  plsc.subcore_barrier()
  pltpu.sync_copy(agg.at[pl.ds(rows0, ROWS_PT)],
                  out_hbm.at[c].at[pl.ds(rows0, ROWS_PT)])


def _sc_aggregate(src, dst, cellw, f0, f1, f2, x_flat, w2_flat):
  mesh = plsc.VectorSubcoreMesh(core_axis_name="c", subcore_axis_name="s")
  f = pl.kernel(
      _sc_body,
      out_type=jax.ShapeDtypeStruct((NC, NPAD, OUT), jnp.float32),
      mesh=mesh,
      scratch_types=(
          [pltpu.VMEM((K ** 3 * OUT,), jnp.float32)] +
          [pltpu.VMEM((CHUNK,), jnp.int32)] * 2 +
          [pltpu.VMEM((SCAT_ROWS, SCAT_C), jnp.int32)] * 2 +
          [pltpu.VMEM((CHUNK,), jnp.int32)] * 2 +
          [pltpu.VMEM((3, CHUNK), jnp.float32)] * 2 +
          [pltpu.VMEM((CHUNK,), jnp.float32)] * 2 +
          [pltpu.VMEM((CHUNK, OUT), jnp.float32)] * 2 +
          [pltpu.SemaphoreType.DMA] * 5 +
          [pltpu.VMEM_SHARED((N,), jnp.float32),
           pltpu.VMEM_SHARED((NPAD, OUT), jnp.float32)]
      ),
      compiler_params=pltpu.CompilerParams(needs_layout_passes=False,
                                           use_tc_tiling_on_sc=False),
  )
  return f(src, dst, cellw, f0, f1, f2, x_flat, w2_flat)


EB = 25600
XB = 10240
NXPAD = 102400


def _prep_body(ei_ref, p_ref, src_ref, dst_ref, cw_ref,
               f0_ref, f1_ref, f2_ref):
  src_ref[...] = ei_ref[0, :]
  dst_ref[...] = ei_ref[1, :]
  pt = p_ref[...].T
  cw = jnp.zeros((EB,), jnp.int32)
  fs = [f0_ref, f1_ref, f2_ref]
  strides = (1, K, K * K)
  for d in range(3):
    pd = pt[d, :] * (K - 1.0)
    lo = jnp.minimum(pd.astype(jnp.int32), K - 2)
    fs[d][...] = pd - lo.astype(jnp.float32)
    cw = cw + lo * (strides[d] * OUT)
  cw_ref[...] = cw


def _x_prep_body(x_ref, o_ref):
  o_ref[...] = x_ref[...].T[0, :]


def _prep(edge_index, pseudo, x):
  src, dst, cellw, f0, f1, f2 = pl.pallas_call(
      _prep_body,
      grid=(E // EB,),
      in_specs=[pl.BlockSpec((2, EB), lambda i: (0, i)),
                pl.BlockSpec((EB, 3), lambda i: (i, 0))],
      out_specs=[pl.BlockSpec((EB,), lambda i: (i,)) for _ in range(6)],
      out_shape=[jax.ShapeDtypeStruct((E,), jnp.int32)] * 3 +
                [jax.ShapeDtypeStruct((E,), jnp.float32)] * 3,
  )(edge_index, pseudo)
  x_flat = pl.pallas_call(
      _x_prep_body,
      grid=(NXPAD // XB,),
      in_specs=[pl.BlockSpec((XB, 1), lambda i: (i, 0))],
      out_specs=pl.BlockSpec((XB,), lambda i: (i,)),
      out_shape=jax.ShapeDtypeStruct((NXPAD,), jnp.float32),
  )(x)
  return src, dst, cellw, f0, f1, f2, x_flat


def _head_body(p0_ref, p1_ref, x_ref, wr_ref, b_ref, lw_ref, lb_ref, o_ref):
  h = p0_ref[...] + p1_ref[...] + x_ref[...] * wr_ref[...] + b_ref[...]
  h = jnp.where(h > 0, h, jnp.exp(jnp.minimum(h, 0.0)) - 1.0)
  q = jnp.dot(h, lw_ref[...], preferred_element_type=jnp.float32) + lb_ref[...]
  sq = jnp.sum(q * q, axis=-1, keepdims=True)
  o_ref[...] = q / (jnp.sqrt(sq) + 1e-4)


def _head(p0, p1, x, w_root, bias, lin_w, lin_b):
  blk = 2000
  grid = (N // blk,)
  return pl.pallas_call(
      _head_body,
      grid=grid,
      in_specs=[
          pl.BlockSpec((blk, OUT), lambda i: (i, 0)),
          pl.BlockSpec((blk, OUT), lambda i: (i, 0)),
          pl.BlockSpec((blk, 1), lambda i: (i, 0)),
          pl.BlockSpec((1, OUT), lambda i: (0, 0)),
          pl.BlockSpec((1, OUT), lambda i: (0, 0)),
          pl.BlockSpec((OUT, 4), lambda i: (0, 0)),
          pl.BlockSpec((1, 4), lambda i: (0, 0)),
      ],
      out_specs=pl.BlockSpec((blk, 4), lambda i: (i, 0)),
      out_shape=jax.ShapeDtypeStruct((N, 4), jnp.float32),
  )(p0, p1, x, w_root, bias, lin_w, lin_b)


@jax.jit
def _run(x, edge_index, pseudo, W, W_root, bias, lin_W, lin_b):
  src, dst, cellw, f0, f1, f2, x_flat = _prep(edge_index, pseudo, x)
  w2_flat = W.reshape(-1)
  partials = _sc_aggregate(src, dst, cellw, f0, f1, f2, x_flat, w2_flat)
  out = _head(partials[0, :N], partials[1, :N], x,
              W_root.reshape(1, OUT), bias.reshape(1, OUT),
              lin_W, lin_b.reshape(1, 4))
  return out.reshape(N, 1, 4)


def kernel(x, edge_index, pseudo, W, W_root, bias, lin_W, lin_b):
  return _run(x, edge_index, pseudo, W, W_root, bias, lin_W, lin_b)
